# Initial kernel scaffold; baseline (speedup 1.0000x reference)
import functools

import jax
import jax.numpy as jnp
from jax import lax
from jax.experimental import pallas as pl
from jax.experimental.pallas import tpu as pltpu
from jax.experimental.pallas import tpu_sc as plsc

B, F, D = 16384, 100, 1000000
NC, NS = 2, 16
NW = NC * NS
S = B // NW
K = F * S
CH = 128
NCH = K // CH
LANES = 16


def _wide_body(idx_hbm, w_hbm, b_hbm, out_hbm, idx_v, vals_v, out_v, b_v, sem):
    c = lax.axis_index("c")
    s = lax.axis_index("s")
    wid = s * NC + c

    pltpu.sync_copy(idx_hbm.at[wid], idx_v)
    pltpu.sync_copy(b_hbm, b_v)

    pltpu.async_copy(w_hbm.at[idx_v], vals_v, sem).wait()

    def red_body(f, accs):
        new = []
        for q in range(4):
            row = 4 * f + q
            for cb in range(8):
                v = vals_v[row, pl.ds(cb * LANES, LANES)]
                new.append(accs[q * 8 + cb] + v)
        return tuple(new)

    zero = jnp.zeros((LANES,), jnp.float32)
    accs = lax.fori_loop(0, F, red_body, tuple(zero for _ in range(32)))

    bvec = b_v[...]
    for q in range(4):
        for cb in range(8):
            z = accs[q * 8 + cb] + bvec
            z = jnp.clip(z, -35.0, 35.0)
            y = 1.0 / (1.0 + jnp.exp(-z))
            out_v[pl.ds(q * 128 + cb * LANES, LANES)] = y

    pltpu.sync_copy(out_v, out_hbm.at[pl.ds(wid * S, S)])


@jax.jit
def _wide_forward(idx, w, b_arr):
    mesh = plsc.VectorSubcoreMesh(core_axis_name="c", subcore_axis_name="s")
    return pl.kernel(
        _wide_body,
        out_type=jax.ShapeDtypeStruct((B,), jnp.float32),
        mesh=mesh,
        scratch_types=[
            pltpu.VMEM((NCH, CH), jnp.int32),
            pltpu.VMEM((NCH, CH), jnp.float32),
            pltpu.VMEM((S,), jnp.float32),
            pltpu.VMEM((LANES,), jnp.float32),
            pltpu.SemaphoreType.DMA,
        ],
    )(idx, w, b_arr)


def kernel(X_w_indices, X_d, y_pred, y, w, b):
    idx = X_w_indices.astype(jnp.int32)
    idx = idx.reshape(NW, S, F).transpose(0, 2, 1).reshape(NW, NCH, CH)
    b_arr = jnp.broadcast_to(b.astype(jnp.float32), (LANES,))
    return _wide_forward(idx, w, b_arr)


# speedup vs baseline: 1.4249x; 1.4249x over previous
import functools

import jax
import jax.numpy as jnp
from jax import lax
from jax.experimental import pallas as pl
from jax.experimental.pallas import tpu as pltpu
from jax.experimental.pallas import tpu_sc as plsc

B, F, D = 16384, 100, 1000000
NC, NS = 2, 16
NW = NC * NS
S = B // NW
K = F * S
CH = 128
NCH = K // CH
LANES = 16


def _wide_body(idx_hbm, w_hbm, b_hbm, out_hbm, idx_v, vals_v, out_v, b_v, sem):
    c = lax.axis_index("c")
    s = lax.axis_index("s")
    wid = s * NC + c

    pltpu.sync_copy(idx_hbm.at[wid], idx_v)
    pltpu.sync_copy(b_hbm, b_v)

    pltpu.async_copy(w_hbm.at[idx_v], vals_v, sem).wait()

    def red_body(f, accs):
        base = f * S
        new = []
        for sb in range(32):
            v = vals_v[pl.ds(base + sb * LANES, LANES)]
            new.append(accs[sb] + v)
        return tuple(new)

    zero = jnp.zeros((LANES,), jnp.float32)
    accs = lax.fori_loop(0, F, red_body, tuple(zero for _ in range(32)))

    bvec = b_v[...]
    for sb in range(32):
        z = accs[sb] + bvec
        z = jnp.clip(z, -35.0, 35.0)
        y = 1.0 / (1.0 + jnp.exp(-z))
        out_v[pl.ds(sb * LANES, LANES)] = y

    pltpu.sync_copy(out_v, out_hbm.at[pl.ds(wid * S, S)])


@jax.jit
def _wide_forward(idx, w, b_arr):
    mesh = plsc.VectorSubcoreMesh(core_axis_name="c", subcore_axis_name="s")
    return pl.kernel(
        _wide_body,
        out_type=jax.ShapeDtypeStruct((B,), jnp.float32),
        mesh=mesh,
        scratch_types=[
            pltpu.VMEM((K,), jnp.int32),
            pltpu.VMEM((K,), jnp.float32),
            pltpu.VMEM((S,), jnp.float32),
            pltpu.VMEM((LANES,), jnp.float32),
            pltpu.SemaphoreType.DMA,
        ],
    )(idx, w, b_arr)


def kernel(X_w_indices, X_d, y_pred, y, w, b):
    idx = X_w_indices.astype(jnp.int32)
    idx = idx.reshape(NW, S, F).transpose(0, 2, 1).reshape(NW, K)
    b_arr = jnp.broadcast_to(b.astype(jnp.float32), (LANES,))
    return _wide_forward(idx, w, b_arr)


# speedup vs baseline: 1.4699x; 1.0316x over previous
import jax
import jax.numpy as jnp
from jax import lax
from jax.experimental import pallas as pl
from jax.experimental.pallas import tpu as pltpu
from jax.experimental.pallas import tpu_sc as plsc

B, F, D = 16384, 100, 1000000
NC, NS = 2, 16
NW = NC * NS
S = B // NW
K = F * S
LANES = 16
NCHK = 8
CS = S // NCHK
CE = CS * F
SB = CS // LANES


def _wide_body(idx_hbm, w_hbm, b_hbm, out_hbm, idx_v, vals_v, out_v, b_v,
               isem, sem0, sem1):
    c = lax.axis_index("c")
    s = lax.axis_index("s")
    wid = s * NC + c
    row = idx_hbm.at[wid]

    pltpu.async_copy(row.at[pl.ds(0, CE)], idx_v.at[pl.ds(0, CE)], isem).wait()
    rest = pltpu.async_copy(row.at[pl.ds(CE, K - CE)],
                            idx_v.at[pl.ds(CE, K - CE)], isem)
    pltpu.sync_copy(b_hbm, b_v)

    sems = (sem0, sem1)

    def fire(chunk):
        lo = chunk * CE
        return pltpu.async_copy(w_hbm.at[idx_v.at[pl.ds(lo, CE)]],
                                vals_v.at[pl.ds(lo, CE)], sems[chunk % 2])

    lane_f = lax.iota(jnp.int32, LANES) * F

    def reduce_chunk(chunk):
        accs = []
        for b_i in range(SB):
            base = (chunk * CS + b_i * LANES) * F
            def body(f, carry):
                acc, idxv = carry
                acc = acc + plsc.load_gather(vals_v, [idxv])
                return (acc, idxv + 1)
            acc, _ = lax.fori_loop(0, F, body,
                                   (jnp.zeros((LANES,), jnp.float32),
                                    lane_f + base))
            accs.append(acc)
        bvec = b_v[...]
        for b_i in range(SB):
            z = accs[b_i] + bvec
            z = jnp.clip(z, -35.0, 35.0)
            y = 1.0 / (1.0 + jnp.exp(-z))
            out_v[pl.ds(chunk * CS + b_i * LANES, LANES)] = y

    inflight = fire(0)
    rest.wait()
    for chunk in range(NCHK):
        nxt = fire(chunk + 1) if chunk + 1 < NCHK else None
        inflight.wait()
        reduce_chunk(chunk)
        inflight = nxt

    pltpu.sync_copy(out_v, out_hbm.at[pl.ds(wid * S, S)])


@jax.jit
def _wide_forward(idx, w, b_arr):
    mesh = plsc.VectorSubcoreMesh(core_axis_name="c", subcore_axis_name="s")
    return pl.kernel(
        _wide_body,
        out_type=jax.ShapeDtypeStruct((B,), jnp.float32),
        mesh=mesh,
        scratch_types=[
            pltpu.VMEM((K,), jnp.int32),
            pltpu.VMEM((K,), jnp.float32),
            pltpu.VMEM((S,), jnp.float32),
            pltpu.VMEM((LANES,), jnp.float32),
            pltpu.SemaphoreType.DMA,
            pltpu.SemaphoreType.DMA,
            pltpu.SemaphoreType.DMA,
        ],
        compiler_params=pltpu.CompilerParams(needs_layout_passes=False),
    )(idx, w, b_arr)


def kernel(X_w_indices, X_d, y_pred, y, w, b):
    idx = X_w_indices.astype(jnp.int32).reshape(NW, K)
    b_arr = jnp.broadcast_to(b.astype(jnp.float32), (LANES,))
    return _wide_forward(idx, w, b_arr)


# speedup vs baseline: 1.9569x; 1.3313x over previous
import jax
import jax.numpy as jnp
from jax import lax
from jax.experimental import pallas as pl
from jax.experimental.pallas import tpu as pltpu
from jax.experimental.pallas import tpu_sc as plsc

B, F, D = 16384, 100, 1000000
NC, NS = 2, 16
NW = NC * NS
S = B // NW
K = F * S
LANES = 16
NCHK = 8
CS = S // NCHK
CE = CS * F
SB = CS // LANES
CE_H = 3200
CE_S = CE - CE_H


def _wide_body(idx_hbm, w_hbm, b_hbm, out_hbm, idx_v, vals_v, out_v, b_v,
               w_sh, isem0, isem1, hsem0, hsem1, ssem0, ssem1, wsem):
    c = lax.axis_index("c")
    s = lax.axis_index("s")
    wid = s * NC + c
    row = idx_hbm.at[wid]

    wcopy = pltpu.make_async_copy(w_hbm, w_sh, wsem)

    @pl.when(s == 0)
    def _():
        wcopy.start()

    isems = (isem0, isem1)
    hsems = (hsem0, hsem1)
    ssems = (ssem0, ssem1)

    def fire_idx(chunk):
        p = chunk % 2
        return pltpu.async_copy(row.at[pl.ds(chunk * CE, CE)],
                                idx_v.at[pl.ds(p * CE, CE)], isems[p])

    def fire_h(chunk):
        p = chunk % 2
        return pltpu.async_copy(
            w_hbm.at[idx_v.at[pl.ds(p * CE, CE_H)]],
            vals_v.at[pl.ds(p * CE, CE_H)], hsems[p])

    def fire_s(chunk):
        p = chunk % 2
        return pltpu.async_copy(
            w_sh.at[idx_v.at[pl.ds(p * CE + CE_H, CE_S)]],
            vals_v.at[pl.ds(p * CE + CE_H, CE_S)], ssems[p])

    lane_f = lax.iota(jnp.int32, LANES) * F

    def reduce_chunk(chunk):
        p = chunk % 2

        def body(f, carry):
            accs, idxvs = carry
            accs = tuple(accs[i] + plsc.load_gather(vals_v, [idxvs[i]])
                         for i in range(SB))
            idxvs = tuple(iv + 1 for iv in idxvs)
            return (accs, idxvs)

        init = (tuple(jnp.zeros((LANES,), jnp.float32) for _ in range(SB)),
                tuple(lane_f + (p * CS + i * LANES) * F for i in range(SB)))
        accs, _ = lax.fori_loop(0, F, body, init)
        bvec = b_v[...]
        for b_i in range(SB):
            z = accs[b_i] + bvec
            z = jnp.clip(z, -35.0, 35.0)
            y = 1.0 / (1.0 + jnp.exp(-z))
            out_v[pl.ds(chunk * CS + b_i * LANES, LANES)] = y

    pltpu.sync_copy(b_hbm, b_v)
    idx_pend = [fire_idx(0), fire_idx(1)]
    idx_pend[0].wait()

    @pl.when(s == 0)
    def _():
        wcopy.wait()
    plsc.subcore_barrier()

    g_pend = [None, None]
    g_pend[0] = (fire_h(0), fire_s(0))
    for chunk in range(NCHK):
        p = chunk % 2
        for d in g_pend[p]:
            d.wait()
        if chunk + 2 < NCHK:
            idx_pend[p] = fire_idx(chunk + 2)
        if chunk + 1 < NCHK:
            idx_pend[(chunk + 1) % 2].wait()
            g_pend[(chunk + 1) % 2] = (fire_h(chunk + 1), fire_s(chunk + 1))
        reduce_chunk(chunk)

    pltpu.sync_copy(out_v, out_hbm.at[pl.ds(wid * S, S)])


@jax.jit
def _wide_forward(idx, w, b_arr):
    mesh = plsc.VectorSubcoreMesh(core_axis_name="c", subcore_axis_name="s")
    return pl.kernel(
        _wide_body,
        out_type=jax.ShapeDtypeStruct((B,), jnp.float32),
        mesh=mesh,
        scratch_types=[
            pltpu.VMEM((2 * CE,), jnp.int32),
            pltpu.VMEM((2 * CE,), jnp.float32),
            pltpu.VMEM((S,), jnp.float32),
            pltpu.VMEM((LANES,), jnp.float32),
            pltpu.VMEM_SHARED((D,), jnp.float32),
            pltpu.SemaphoreType.DMA,
            pltpu.SemaphoreType.DMA,
            pltpu.SemaphoreType.DMA,
            pltpu.SemaphoreType.DMA,
            pltpu.SemaphoreType.DMA,
            pltpu.SemaphoreType.DMA,
            pltpu.SemaphoreType.DMA,
        ],
        compiler_params=pltpu.CompilerParams(needs_layout_passes=False),
    )(idx, w, b_arr)


def kernel(X_w_indices, X_d, y_pred, y, w, b):
    idx = X_w_indices.astype(jnp.int32).reshape(NW, K)
    b_arr = jnp.broadcast_to(b.astype(jnp.float32), (LANES,))
    return _wide_forward(idx, w, b_arr)


# speedup vs baseline: 2.0755x; 1.0606x over previous
import jax
import jax.numpy as jnp
from jax import lax
from jax.experimental import pallas as pl
from jax.experimental.pallas import tpu as pltpu
from jax.experimental.pallas import tpu_sc as plsc

B, F, D = 16384, 100, 1000000
NC, NS = 2, 16
NW = NC * NS
S = B // NW
K = F * S
LANES = 16
NCHK = 8
CS = S // NCHK
CE = CS * F
SB = CS // LANES
CE_H = 2816
CE_S = CE - CE_H


def _wide_body(idx_hbm, w_hbm, b_hbm, out_hbm, idx_v, vals_v, out_v, b_v,
               w_sh, isem0, isem1, hsem0, hsem1, ssem0, ssem1, wsem):
    c = lax.axis_index("c")
    s = lax.axis_index("s")
    wid = s * NC + c
    row = idx_hbm.at[wid]

    wcopy = pltpu.make_async_copy(w_hbm, w_sh, wsem)

    @pl.when(s == 0)
    def _():
        wcopy.start()

    isems = (isem0, isem1)
    hsems = (hsem0, hsem1)
    ssems = (ssem0, ssem1)

    def fire_idx(chunk):
        p = chunk % 2
        return pltpu.async_copy(row.at[pl.ds(chunk * CE, CE)],
                                idx_v.at[pl.ds(p * CE, CE)], isems[p])

    def fire_h(chunk):
        p = chunk % 2
        return pltpu.async_copy(
            w_hbm.at[idx_v.at[pl.ds(p * CE, CE_H)]],
            vals_v.at[pl.ds(p * CE, CE_H)], hsems[p])

    def fire_s(chunk):
        p = chunk % 2
        return pltpu.async_copy(
            w_sh.at[idx_v.at[pl.ds(p * CE + CE_H, CE_S)]],
            vals_v.at[pl.ds(p * CE + CE_H, CE_S)], ssems[p])

    lane_f = lax.iota(jnp.int32, LANES) * F

    def reduce_chunk(chunk):
        p = chunk % 2

        def body(f, carry):
            accs, idxvs = carry
            accs = tuple(accs[i] + plsc.load_gather(vals_v, [idxvs[i]])
                         for i in range(SB))
            idxvs = tuple(iv + 1 for iv in idxvs)
            return (accs, idxvs)

        init = (tuple(jnp.zeros((LANES,), jnp.float32) for _ in range(SB)),
                tuple(lane_f + (p * CS + i * LANES) * F for i in range(SB)))
        accs, _ = lax.fori_loop(0, F, body, init)
        bvec = b_v[...]
        for b_i in range(SB):
            z = accs[b_i] + bvec
            z = jnp.clip(z, -35.0, 35.0)
            y = 1.0 / (1.0 + jnp.exp(-z))
            out_v[pl.ds(chunk * CS + b_i * LANES, LANES)] = y

    pltpu.sync_copy(b_hbm, b_v)
    idx_pend = [fire_idx(0), fire_idx(1)]
    idx_pend[0].wait()
    gh0 = fire_h(0)

    @pl.when(s == 0)
    def _():
        wcopy.wait()
    plsc.subcore_barrier()

    g_pend = [None, None]
    g_pend[0] = (gh0, fire_s(0))
    for chunk in range(NCHK):
        p = chunk % 2
        for d in g_pend[p]:
            d.wait()
        if chunk + 2 < NCHK:
            idx_pend[p] = fire_idx(chunk + 2)
        if chunk + 1 < NCHK:
            idx_pend[(chunk + 1) % 2].wait()
            g_pend[(chunk + 1) % 2] = (fire_h(chunk + 1), fire_s(chunk + 1))
        reduce_chunk(chunk)

    pltpu.sync_copy(out_v, out_hbm.at[pl.ds(wid * S, S)])


@jax.jit
def _wide_forward(idx, w, b_arr):
    mesh = plsc.VectorSubcoreMesh(core_axis_name="c", subcore_axis_name="s")
    return pl.kernel(
        _wide_body,
        out_type=jax.ShapeDtypeStruct((B,), jnp.float32),
        mesh=mesh,
        scratch_types=[
            pltpu.VMEM((2 * CE,), jnp.int32),
            pltpu.VMEM((2 * CE,), jnp.float32),
            pltpu.VMEM((S,), jnp.float32),
            pltpu.VMEM((LANES,), jnp.float32),
            pltpu.VMEM_SHARED((D,), jnp.float32),
            pltpu.SemaphoreType.DMA,
            pltpu.SemaphoreType.DMA,
            pltpu.SemaphoreType.DMA,
            pltpu.SemaphoreType.DMA,
            pltpu.SemaphoreType.DMA,
            pltpu.SemaphoreType.DMA,
            pltpu.SemaphoreType.DMA,
        ],
        compiler_params=pltpu.CompilerParams(needs_layout_passes=False),
    )(idx, w, b_arr)


def kernel(X_w_indices, X_d, y_pred, y, w, b):
    idx = X_w_indices.astype(jnp.int32).reshape(NW, K)
    b_arr = jnp.broadcast_to(b.astype(jnp.float32), (LANES,))
    return _wide_forward(idx, w, b_arr)
